# Initial kernel scaffold; baseline (speedup 1.0000x reference)
import jax
import jax.numpy as jnp
from jax.experimental import pallas as pl


def kernel(z, edge_index, W1, b1, W2, b2, gamma, beta):
    raise NotImplementedError("write your pallas kernel here")


# speedup vs baseline: 11.6347x; 11.6347x over previous
import functools

import jax
import jax.numpy as jnp
from jax import lax
from jax.experimental import pallas as pl
from jax.experimental.pallas import tpu as pltpu
from jax.experimental.pallas import tpu_sc as plsc

D = 128
N = 10000
E = 320000
NP = 10240
NC = 2
NS = 16
NW = NC * NS
EPW = E // NW
CHUNK = 80
NCHUNK = EPW // CHUNK
RPT = NP // NS
BLK = 1000


def _mesh():
    return plsc.VectorSubcoreMesh(
        core_axis_name="c", subcore_axis_name="s", num_cores=NC, num_subcores=NS
    )


def _mlp_body(z_ref, w1_ref, b1_ref, w2_ref, b2_ref, out_ref):
    h = jnp.maximum(
        jnp.dot(z_ref[...], w1_ref[...], preferred_element_type=jnp.float32)
        + b1_ref[...],
        0.0,
    )
    out_ref[...] = (
        jnp.dot(h, w2_ref[...], preferred_element_type=jnp.float32) + b2_ref[...]
    )


def _mlp(z, W1, b1, W2, b2):
    return pl.pallas_call(
        _mlp_body,
        grid=(N // BLK,),
        in_specs=[
            pl.BlockSpec((BLK, D), lambda i: (i, 0)),
            pl.BlockSpec((D, D), lambda i: (0, 0)),
            pl.BlockSpec((1, D), lambda i: (0, 0)),
            pl.BlockSpec((D, D), lambda i: (0, 0)),
            pl.BlockSpec((1, D), lambda i: (0, 0)),
        ],
        out_specs=pl.BlockSpec((BLK, D), lambda i: (i, 0)),
        out_shape=jax.ShapeDtypeStruct((N, D), jnp.float32),
    )(z, W1, b1, W2, b2)


def _scatter_body(
    m_hbm, src_hbm, dst_hbm, znd_hbm, zn_hbm,
    agg_out, deg_out,
    src_v, dst_v, rows_v, ones_v, agg_sh, deg_sh, sem,
):
    cid = lax.axis_index("c")
    sid = lax.axis_index("s")
    wid = sid * NC + cid

    pltpu.sync_copy(src_hbm.at[wid], src_v)
    pltpu.sync_copy(dst_hbm.at[wid], dst_v)
    pltpu.sync_copy(znd_hbm.at[pl.ds(sid * RPT, RPT)], agg_sh.at[pl.ds(sid * RPT, RPT)])
    pltpu.sync_copy(zn_hbm.at[pl.ds(sid * RPT, RPT)], deg_sh.at[pl.ds(sid * RPT, RPT)])
    for i in range(CHUNK // 16):
        ones_v[pl.ds(i * 16, 16)] = jnp.full((16,), 1.0, jnp.float32)
    plsc.subcore_barrier()

    @pl.loop(0, NCHUNK)
    def _chunk(j):
        pltpu.async_copy(m_hbm.at[dst_v.at[j]], rows_v, sem).wait()
        pltpu.sync_copy(rows_v, agg_sh.at[src_v.at[j]], add=True)
        pltpu.sync_copy(ones_v, deg_sh.at[src_v.at[j]], add=True)

    plsc.subcore_barrier()
    pltpu.sync_copy(
        agg_sh.at[pl.ds(sid * RPT, RPT)], agg_out.at[cid, pl.ds(sid * RPT, RPT)]
    )
    pltpu.sync_copy(
        deg_sh.at[pl.ds(sid * RPT, RPT)], deg_out.at[cid, pl.ds(sid * RPT, RPT)]
    )


def _scatter(m, src3, dst3, zeros_nd, zeros_n):
    f = pl.kernel(
        _scatter_body,
        out_type=(
            jax.ShapeDtypeStruct((NC, NP, D), jnp.float32),
            jax.ShapeDtypeStruct((NC, NP), jnp.float32),
        ),
        mesh=_mesh(),
        scratch_types=[
            pltpu.VMEM((NCHUNK, CHUNK), jnp.int32),
            pltpu.VMEM((NCHUNK, CHUNK), jnp.int32),
            pltpu.VMEM((CHUNK, D), jnp.float32),
            pltpu.VMEM((CHUNK,), jnp.float32),
            pltpu.VMEM_SHARED((NP, D), jnp.float32),
            pltpu.VMEM_SHARED((NP,), jnp.float32),
            pltpu.SemaphoreType.DMA,
        ],
    )
    return f(m, src3, dst3, zeros_nd, zeros_n)


def _alpha_body(deg_hbm, src_hbm, alpha_hbm, d0_v, d1_v, rdeg_v, src_v, alpha_v):
    cid = lax.axis_index("c")
    sid = lax.axis_index("s")
    wid = sid * NC + cid

    pltpu.sync_copy(deg_hbm.at[0], d0_v)
    pltpu.sync_copy(deg_hbm.at[1], d1_v)
    pltpu.sync_copy(src_hbm.at[pl.ds(wid * EPW, EPW)], src_v)

    @pl.loop(0, NP // 16)
    def _rd(i):
        o = pl.ds(i * 16, 16)
        rdeg_v[o] = 1.0 / (d0_v[o] + d1_v[o] + 1e-12)

    @pl.loop(0, EPW // 16)
    def _ga(j):
        o = pl.ds(j * 16, 16)
        alpha_v[o] = plsc.load_gather(rdeg_v, [src_v[o]])

    pltpu.sync_copy(alpha_v, alpha_hbm.at[pl.ds(wid * EPW, EPW)])


def _alpha(deg_p, src):
    f = pl.kernel(
        _alpha_body,
        out_type=jax.ShapeDtypeStruct((E,), jnp.float32),
        mesh=_mesh(),
        scratch_types=[
            pltpu.VMEM((NP,), jnp.float32),
            pltpu.VMEM((NP,), jnp.float32),
            pltpu.VMEM((NP,), jnp.float32),
            pltpu.VMEM((EPW,), jnp.int32),
            pltpu.VMEM((EPW,), jnp.float32),
        ],
        compiler_params=pltpu.CompilerParams(needs_layout_passes=False),
    )
    return f(deg_p, src)


LBLK = 1024


def _ln_body(z_ref, a0_ref, a1_ref, d0_ref, d1_ref, g_ref, b_ref, out_ref):
    d = d0_ref[...] + d1_ref[...]
    rdeg = 1.0 / (d + 1e-12)
    x = z_ref[...] + (a0_ref[...] + a1_ref[...]) * rdeg
    mean = jnp.mean(x, axis=1, keepdims=True)
    xc = x - mean
    var = jnp.mean(xc * xc, axis=1, keepdims=True)
    out_ref[...] = xc * lax.rsqrt(var + 1e-5) * g_ref[...] + b_ref[...]


def _ln(z_pad, agg_p, deg_p, gamma, beta):
    a0, a1 = agg_p[0], agg_p[1]
    d0, d1 = deg_p[0].reshape(NP, 1), deg_p[1].reshape(NP, 1)
    return pl.pallas_call(
        _ln_body,
        grid=(NP // LBLK,),
        in_specs=[
            pl.BlockSpec((LBLK, D), lambda i: (i, 0)),
            pl.BlockSpec((LBLK, D), lambda i: (i, 0)),
            pl.BlockSpec((LBLK, D), lambda i: (i, 0)),
            pl.BlockSpec((LBLK, 1), lambda i: (i, 0)),
            pl.BlockSpec((LBLK, 1), lambda i: (i, 0)),
            pl.BlockSpec((1, D), lambda i: (0, 0)),
            pl.BlockSpec((1, D), lambda i: (0, 0)),
        ],
        out_specs=pl.BlockSpec((LBLK, D), lambda i: (i, 0)),
        out_shape=jax.ShapeDtypeStruct((NP, D), jnp.float32),
    )(z_pad, a0, a1, d0, d1, gamma, beta)


def kernel(z, edge_index, W1, b1, W2, b2, gamma, beta):
    ei = edge_index.astype(jnp.int32)
    src = ei[0]
    dst = ei[1]
    src3 = src.reshape(NW, NCHUNK, CHUNK)
    dst3 = dst.reshape(NW, NCHUNK, CHUNK)
    zeros_nd = jnp.zeros((NP, D), jnp.float32)
    zeros_n = jnp.zeros((NP,), jnp.float32)

    m = _mlp(z, W1, b1.reshape(1, D), W2, b2.reshape(1, D))
    agg_p, deg_p = _scatter(m, src3, dst3, zeros_nd, zeros_n)
    alpha = _alpha(deg_p, src)
    z_pad = jnp.pad(z, ((0, NP - N), (0, 0)))
    out = _ln(z_pad, agg_p, deg_p, gamma.reshape(1, D), beta.reshape(1, D))
    return (out[:N], alpha)


# speedup vs baseline: 12.1938x; 1.0481x over previous
import jax
import jax.numpy as jnp
from jax import lax
from jax.experimental import pallas as pl
from jax.experimental.pallas import tpu as pltpu
from jax.experimental.pallas import tpu_sc as plsc

D = 128
N = 10000
E = 320000
NP = 10240
NC = 2
NS = 16
NW = NC * NS
EPW = E // NW
CHUNK = 80
NCHUNK = EPW // CHUNK
RPT = NP // NS
EPT = E // NS
BLK = 1000
LBLK = 1024


def _mesh():
    return plsc.VectorSubcoreMesh(
        core_axis_name="c", subcore_axis_name="s", num_cores=NC, num_subcores=NS
    )


DH = D // 2


def _mlp_body(z_ref, w1_ref, b1_ref, w2_ref, b2_ref, out0_ref, out1_ref):
    h = jnp.maximum(
        jnp.dot(z_ref[...], w1_ref[...], preferred_element_type=jnp.float32)
        + b1_ref[...],
        0.0,
    )
    m = jnp.dot(h, w2_ref[...], preferred_element_type=jnp.float32) + b2_ref[...]
    out0_ref[...] = m[:, :DH]
    out1_ref[...] = m[:, DH:]


def _mlp(z, W1, b1, W2, b2):
    return pl.pallas_call(
        _mlp_body,
        grid=(N // BLK,),
        in_specs=[
            pl.BlockSpec((BLK, D), lambda i: (i, 0)),
            pl.BlockSpec((D, D), lambda i: (0, 0)),
            pl.BlockSpec((1, D), lambda i: (0, 0)),
            pl.BlockSpec((D, D), lambda i: (0, 0)),
            pl.BlockSpec((1, D), lambda i: (0, 0)),
        ],
        out_specs=[
            pl.BlockSpec((BLK, DH), lambda i: (i, 0)),
            pl.BlockSpec((BLK, DH), lambda i: (i, 0)),
        ],
        out_shape=[
            jax.ShapeDtypeStruct((N, DH), jnp.float32),
            jax.ShapeDtypeStruct((N, DH), jnp.float32),
        ],
    )(z, W1, b1, W2, b2)


def _deg_body(
    src_hbm, alpha_hbm, rdeg_hbm,
    ldeg_v, srch_v, loc_v, rdegf_v, alpha_v, hist_sh, rdeg_sh,
):
    cid = lax.axis_index("c")
    sid = lax.axis_index("s")
    wid = sid * NC + cid

    @pl.loop(0, NP // 16)
    def _z(i):
        ldeg_v[pl.ds(i * 16, 16)] = jnp.zeros((16,), jnp.float32)

    pltpu.sync_copy(src_hbm.at[pl.ds(sid * EPT, EPT)], srch_v)
    ones16 = jnp.full((16,), 1.0, jnp.float32)

    @pl.loop(0, EPT // 16)
    def _h(k):
        idx = srch_v[pl.ds(k * 16, 16)]
        plsc.addupdate_scatter(ldeg_v, [idx], ones16)

    pltpu.sync_copy(ldeg_v, hist_sh.at[sid])
    plsc.subcore_barrier()
    pltpu.sync_copy(hist_sh.at[:, pl.ds(sid * RPT, RPT)], loc_v)

    @pl.loop(0, RPT // 16)
    def _r(c):
        o = pl.ds(c * 16, 16)
        acc = loc_v[0, o]
        for r in range(1, NS):
            acc = acc + loc_v[r, o]
        ldeg_v[o] = 1.0 / (acc + 1e-12)

    pltpu.sync_copy(ldeg_v.at[pl.ds(0, RPT)], rdeg_sh.at[pl.ds(sid * RPT, RPT)])
    plsc.subcore_barrier()

    pltpu.sync_copy(rdeg_sh, rdegf_v)

    @pl.when(cid == 0)
    def _():
        pltpu.sync_copy(rdegf_v.at[pl.ds(sid * RPT, RPT)], rdeg_hbm.at[pl.ds(sid * RPT, RPT)])

    pltpu.sync_copy(src_hbm.at[pl.ds(wid * EPW, EPW)], srch_v.at[pl.ds(0, EPW)])

    @pl.loop(0, EPW // 16)
    def _g(j):
        o = pl.ds(j * 16, 16)
        alpha_v[o] = plsc.load_gather(rdegf_v, [srch_v[o]])

    pltpu.sync_copy(alpha_v, alpha_hbm.at[pl.ds(wid * EPW, EPW)])


def _deg_alpha(src):
    f = pl.kernel(
        _deg_body,
        out_type=(
            jax.ShapeDtypeStruct((E,), jnp.float32),
            jax.ShapeDtypeStruct((NP,), jnp.float32),
        ),
        mesh=_mesh(),
        scratch_types=[
            pltpu.VMEM((NP,), jnp.float32),
            pltpu.VMEM((EPT,), jnp.int32),
            pltpu.VMEM((NS, RPT), jnp.float32),
            pltpu.VMEM((NP,), jnp.float32),
            pltpu.VMEM((EPW,), jnp.float32),
            pltpu.VMEM_SHARED((NS, NP), jnp.float32),
            pltpu.VMEM_SHARED((NP,), jnp.float32),
        ],
        compiler_params=pltpu.CompilerParams(needs_layout_passes=False),
    )
    return f(src)


NCH2 = EPT // CHUNK


def _scatter_body(
    m0_hbm, m1_hbm, src_hbm, dst_hbm, znd_hbm,
    agg_out,
    src_v, dst_v, rows0_v, rows1_v, agg_sh, sem0, sem1,
):
    cid = lax.axis_index("c")
    sid = lax.axis_index("s")

    pltpu.sync_copy(src_hbm.at[sid], src_v)
    pltpu.sync_copy(dst_hbm.at[sid], dst_v)
    pltpu.sync_copy(znd_hbm.at[pl.ds(sid * RPT, RPT)], agg_sh.at[pl.ds(sid * RPT, RPT)])
    plsc.subcore_barrier()

    def _run(m_hbm):
        @pl.loop(0, NCH2, step=2)
        def _chunk(j):
            c0 = pltpu.async_copy(m_hbm.at[dst_v.at[j]], rows0_v, sem0)
            c1 = pltpu.async_copy(m_hbm.at[dst_v.at[j + 1]], rows1_v, sem1)
            c0.wait()
            pltpu.sync_copy(rows0_v, agg_sh.at[src_v.at[j]], add=True)
            c1.wait()
            pltpu.sync_copy(rows1_v, agg_sh.at[src_v.at[j + 1]], add=True)

    @pl.when(cid == 0)
    def _():
        _run(m0_hbm)

    @pl.when(cid == 1)
    def _():
        _run(m1_hbm)

    plsc.subcore_barrier()
    pltpu.sync_copy(
        agg_sh.at[pl.ds(sid * RPT, RPT)], agg_out.at[cid, pl.ds(sid * RPT, RPT)]
    )


def _scatter(m0, m1, src2, dst2, zeros_nd):
    f = pl.kernel(
        _scatter_body,
        out_type=jax.ShapeDtypeStruct((NC, NP, DH), jnp.float32),
        mesh=_mesh(),
        scratch_types=[
            pltpu.VMEM((NCH2, CHUNK), jnp.int32),
            pltpu.VMEM((NCH2, CHUNK), jnp.int32),
            pltpu.VMEM((CHUNK, DH), jnp.float32),
            pltpu.VMEM((CHUNK, DH), jnp.float32),
            pltpu.VMEM_SHARED((NP, DH), jnp.float32),
            pltpu.SemaphoreType.DMA,
            pltpu.SemaphoreType.DMA,
        ],
        compiler_params=pltpu.CompilerParams(use_tc_tiling_on_sc=False),
    )
    return f(m0, m1, src2, dst2, zeros_nd)


def _ln_body(z_ref, a0_ref, a1_ref, rd_ref, g_ref, b_ref, out_ref):
    agg = jnp.concatenate([a0_ref[...], a1_ref[...]], axis=1)
    x = z_ref[...] + agg * rd_ref[...]
    mean = jnp.mean(x, axis=1, keepdims=True)
    xc = x - mean
    var = jnp.mean(xc * xc, axis=1, keepdims=True)
    out_ref[...] = xc * lax.rsqrt(var + 1e-5) * g_ref[...] + b_ref[...]


def _ln(z_pad, agg_p, rdeg, gamma, beta):
    a0, a1 = agg_p[0], agg_p[1]
    rd = rdeg.reshape(NP, 1)
    return pl.pallas_call(
        _ln_body,
        grid=(NP // LBLK,),
        in_specs=[
            pl.BlockSpec((LBLK, D), lambda i: (i, 0)),
            pl.BlockSpec((LBLK, DH), lambda i: (i, 0)),
            pl.BlockSpec((LBLK, DH), lambda i: (i, 0)),
            pl.BlockSpec((LBLK, 1), lambda i: (i, 0)),
            pl.BlockSpec((1, D), lambda i: (0, 0)),
            pl.BlockSpec((1, D), lambda i: (0, 0)),
        ],
        out_specs=pl.BlockSpec((LBLK, D), lambda i: (i, 0)),
        out_shape=jax.ShapeDtypeStruct((NP, D), jnp.float32),
    )(z_pad, a0, a1, rd, gamma, beta)


def kernel(z, edge_index, W1, b1, W2, b2, gamma, beta):
    ei = edge_index.astype(jnp.int32)
    src = ei[0]
    dst = ei[1]
    src2 = src.reshape(NS, NCH2, CHUNK)
    dst2 = dst.reshape(NS, NCH2, CHUNK)
    zeros_nd = jnp.zeros((NP, DH), jnp.float32)

    m0, m1 = _mlp(z, W1, b1.reshape(1, D), W2, b2.reshape(1, D))
    alpha, rdeg = _deg_alpha(src)
    agg_p = _scatter(m0, m1, src2, dst2, zeros_nd)
    z_pad = jnp.pad(z, ((0, NP - N), (0, 0)))
    out = _ln(z_pad, agg_p, rdeg, gamma.reshape(1, D), beta.reshape(1, D))
    return (out[:N], alpha)


# speedup vs baseline: 12.5107x; 1.0260x over previous
import jax
import jax.numpy as jnp
from jax import lax
from jax.experimental import pallas as pl
from jax.experimental.pallas import tpu as pltpu
from jax.experimental.pallas import tpu_sc as plsc

D = 128
N = 10000
E = 320000
NP = 10240
NC = 2
NS = 16
NW = NC * NS
EPW = E // NW
CHUNK = 80
NCHUNK = EPW // CHUNK
RPT = NP // NS
EPT = E // NS
BLK = 1000
LBLK = 1024


def _mesh():
    return plsc.VectorSubcoreMesh(
        core_axis_name="c", subcore_axis_name="s", num_cores=NC, num_subcores=NS
    )


DH = D // 2


def _mlp_body(z_ref, w1_ref, b1_ref, w2_ref, b2_ref, out0_ref, out1_ref):
    h = jnp.maximum(
        jnp.dot(z_ref[...], w1_ref[...], preferred_element_type=jnp.float32)
        + b1_ref[...],
        0.0,
    )
    m = jnp.dot(h, w2_ref[...], preferred_element_type=jnp.float32) + b2_ref[...]
    out0_ref[...] = m[:, :DH]
    out1_ref[...] = m[:, DH:]


def _mlp(z, W1, b1, W2, b2):
    return pl.pallas_call(
        _mlp_body,
        grid=(N // BLK,),
        in_specs=[
            pl.BlockSpec((BLK, D), lambda i: (i, 0)),
            pl.BlockSpec((D, D), lambda i: (0, 0)),
            pl.BlockSpec((1, D), lambda i: (0, 0)),
            pl.BlockSpec((D, D), lambda i: (0, 0)),
            pl.BlockSpec((1, D), lambda i: (0, 0)),
        ],
        out_specs=[
            pl.BlockSpec((BLK, DH), lambda i: (i, 0)),
            pl.BlockSpec((BLK, DH), lambda i: (i, 0)),
        ],
        out_shape=[
            jax.ShapeDtypeStruct((N, DH), jnp.float32),
            jax.ShapeDtypeStruct((N, DH), jnp.float32),
        ],
    )(z, W1, b1, W2, b2)


def _deg_body(
    src_hbm, alpha_hbm, rdeg_hbm,
    ldeg_v, srch_v, loc_v, rdegf_v, alpha_v, hist_sh, rdeg_sh,
):
    cid = lax.axis_index("c")
    sid = lax.axis_index("s")
    wid = sid * NC + cid

    @pl.loop(0, NP // 16)
    def _z(i):
        ldeg_v[pl.ds(i * 16, 16)] = jnp.zeros((16,), jnp.float32)

    pltpu.sync_copy(src_hbm.at[pl.ds(sid * EPT, EPT)], srch_v)
    ones16 = jnp.full((16,), 1.0, jnp.float32)

    @pl.loop(0, EPT // 16)
    def _h(k):
        idx = srch_v[pl.ds(k * 16, 16)]
        plsc.addupdate_scatter(ldeg_v, [idx], ones16)

    pltpu.sync_copy(ldeg_v, hist_sh.at[sid])
    plsc.subcore_barrier()
    pltpu.sync_copy(hist_sh.at[:, pl.ds(sid * RPT, RPT)], loc_v)

    @pl.loop(0, RPT // 16)
    def _r(c):
        o = pl.ds(c * 16, 16)
        acc = loc_v[0, o]
        for r in range(1, NS):
            acc = acc + loc_v[r, o]
        ldeg_v[o] = 1.0 / (acc + 1e-12)

    pltpu.sync_copy(ldeg_v.at[pl.ds(0, RPT)], rdeg_sh.at[pl.ds(sid * RPT, RPT)])
    plsc.subcore_barrier()

    pltpu.sync_copy(rdeg_sh, rdegf_v)

    @pl.when(cid == 0)
    def _():
        pltpu.sync_copy(rdegf_v.at[pl.ds(sid * RPT, RPT)], rdeg_hbm.at[pl.ds(sid * RPT, RPT)])

    pltpu.sync_copy(src_hbm.at[pl.ds(wid * EPW, EPW)], srch_v.at[pl.ds(0, EPW)])

    @pl.loop(0, EPW // 16)
    def _g(j):
        o = pl.ds(j * 16, 16)
        alpha_v[o] = plsc.load_gather(rdegf_v, [srch_v[o]])

    pltpu.sync_copy(alpha_v, alpha_hbm.at[pl.ds(wid * EPW, EPW)])


def _deg_alpha(src):
    f = pl.kernel(
        _deg_body,
        out_type=(
            jax.ShapeDtypeStruct((E,), jnp.float32),
            jax.ShapeDtypeStruct((NP,), jnp.float32),
        ),
        mesh=_mesh(),
        scratch_types=[
            pltpu.VMEM((NP,), jnp.float32),
            pltpu.VMEM((EPT,), jnp.int32),
            pltpu.VMEM((NS, RPT), jnp.float32),
            pltpu.VMEM((NP,), jnp.float32),
            pltpu.VMEM((EPW,), jnp.float32),
            pltpu.VMEM_SHARED((NS, NP), jnp.float32),
            pltpu.VMEM_SHARED((NP,), jnp.float32),
        ],
        compiler_params=pltpu.CompilerParams(needs_layout_passes=False),
    )
    return f(src)


NCH2 = EPT // CHUNK


def _scatter_body(
    m0_hbm, m1_hbm, src_hbm, dst_hbm, znd_hbm,
    agg_out,
    src_v, dst_v, rows0_v, rows1_v, agg_sh, gs0, gs1, ss0, ss1,
):
    cid = lax.axis_index("c")
    sid = lax.axis_index("s")

    pltpu.sync_copy(src_hbm.at[sid], src_v)
    pltpu.sync_copy(dst_hbm.at[sid], dst_v)
    pltpu.sync_copy(znd_hbm.at[pl.ds(sid * RPT, RPT)], agg_sh.at[pl.ds(sid * RPT, RPT)])
    plsc.subcore_barrier()

    def _run(m_hbm):
        def g_start(j, buf, gs):
            pltpu.async_copy(m_hbm.at[dst_v.at[j]], buf, gs)

        def g_wait(j, buf, gs):
            pltpu.make_async_copy(m_hbm.at[dst_v.at[j]], buf, gs).wait()

        def s_start(j, buf, ss):
            pltpu.async_copy(buf, agg_sh.at[src_v.at[j]], ss, add=True)

        def s_wait(j, buf, ss):
            pltpu.make_async_copy(buf, agg_sh.at[src_v.at[j]], ss).wait()

        g_start(0, rows0_v, gs0)
        g_start(1, rows1_v, gs1)

        @pl.loop(0, NCH2 - 4, step=2)
        def _chunk(j):
            g_wait(j, rows0_v, gs0)
            s_start(j, rows0_v, ss0)
            g_wait(j + 1, rows1_v, gs1)
            s_start(j + 1, rows1_v, ss1)
            s_wait(j, rows0_v, ss0)
            g_start(j + 2, rows0_v, gs0)
            s_wait(j + 1, rows1_v, ss1)
            g_start(j + 3, rows1_v, gs1)

        ja = NCH2 - 4
        g_wait(ja, rows0_v, gs0)
        s_start(ja, rows0_v, ss0)
        g_wait(ja + 1, rows1_v, gs1)
        s_start(ja + 1, rows1_v, ss1)
        s_wait(ja, rows0_v, ss0)
        g_start(ja + 2, rows0_v, gs0)
        s_wait(ja + 1, rows1_v, ss1)
        g_start(ja + 3, rows1_v, gs1)
        g_wait(ja + 2, rows0_v, gs0)
        s_start(ja + 2, rows0_v, ss0)
        g_wait(ja + 3, rows1_v, gs1)
        s_start(ja + 3, rows1_v, ss1)
        s_wait(ja + 2, rows0_v, ss0)
        s_wait(ja + 3, rows1_v, ss1)

    @pl.when(cid == 0)
    def _():
        _run(m0_hbm)

    @pl.when(cid == 1)
    def _():
        _run(m1_hbm)

    plsc.subcore_barrier()
    pltpu.sync_copy(
        agg_sh.at[pl.ds(sid * RPT, RPT)], agg_out.at[cid, pl.ds(sid * RPT, RPT)]
    )


def _scatter(m0, m1, src2, dst2, zeros_nd):
    f = pl.kernel(
        _scatter_body,
        out_type=jax.ShapeDtypeStruct((NC, NP, DH), jnp.float32),
        mesh=_mesh(),
        scratch_types=[
            pltpu.VMEM((NCH2, CHUNK), jnp.int32),
            pltpu.VMEM((NCH2, CHUNK), jnp.int32),
            pltpu.VMEM((CHUNK, DH), jnp.float32),
            pltpu.VMEM((CHUNK, DH), jnp.float32),
            pltpu.VMEM_SHARED((NP, DH), jnp.float32),
            pltpu.SemaphoreType.DMA,
            pltpu.SemaphoreType.DMA,
            pltpu.SemaphoreType.DMA,
            pltpu.SemaphoreType.DMA,
        ],
        compiler_params=pltpu.CompilerParams(use_tc_tiling_on_sc=False),
    )
    return f(m0, m1, src2, dst2, zeros_nd)


def _ln_body(z_ref, a0_ref, a1_ref, rd_ref, g_ref, b_ref, out_ref):
    agg = jnp.concatenate([a0_ref[...], a1_ref[...]], axis=1)
    x = z_ref[...] + agg * rd_ref[...]
    mean = jnp.mean(x, axis=1, keepdims=True)
    xc = x - mean
    var = jnp.mean(xc * xc, axis=1, keepdims=True)
    out_ref[...] = xc * lax.rsqrt(var + 1e-5) * g_ref[...] + b_ref[...]


def _ln(z_pad, agg_p, rdeg, gamma, beta):
    a0, a1 = agg_p[0], agg_p[1]
    rd = rdeg.reshape(NP, 1)
    return pl.pallas_call(
        _ln_body,
        grid=(NP // LBLK,),
        in_specs=[
            pl.BlockSpec((LBLK, D), lambda i: (i, 0)),
            pl.BlockSpec((LBLK, DH), lambda i: (i, 0)),
            pl.BlockSpec((LBLK, DH), lambda i: (i, 0)),
            pl.BlockSpec((LBLK, 1), lambda i: (i, 0)),
            pl.BlockSpec((1, D), lambda i: (0, 0)),
            pl.BlockSpec((1, D), lambda i: (0, 0)),
        ],
        out_specs=pl.BlockSpec((LBLK, D), lambda i: (i, 0)),
        out_shape=jax.ShapeDtypeStruct((NP, D), jnp.float32),
    )(z_pad, a0, a1, rd, gamma, beta)


def kernel(z, edge_index, W1, b1, W2, b2, gamma, beta):
    ei = edge_index.astype(jnp.int32)
    src = ei[0]
    dst = ei[1]
    src2 = src.reshape(NS, NCH2, CHUNK)
    dst2 = dst.reshape(NS, NCH2, CHUNK)
    zeros_nd = jnp.zeros((NP, DH), jnp.float32)

    m0, m1 = _mlp(z, W1, b1.reshape(1, D), W2, b2.reshape(1, D))
    alpha, rdeg = _deg_alpha(src)
    agg_p = _scatter(m0, m1, src2, dst2, zeros_nd)
    z_pad = jnp.pad(z, ((0, NP - N), (0, 0)))
    out = _ln(z_pad, agg_p, rdeg, gamma.reshape(1, D), beta.reshape(1, D))
    return (out[:N], alpha)


# speedup vs baseline: 12.9642x; 1.0362x over previous
import jax
import jax.numpy as jnp
from jax import lax
from jax.experimental import pallas as pl
from jax.experimental.pallas import tpu as pltpu
from jax.experimental.pallas import tpu_sc as plsc

D = 128
N = 10000
E = 320000
NP = 10240
NC = 2
NS = 16
NW = NC * NS
EPW = E // NW
CHUNK = 80
NCHUNK = EPW // CHUNK
RPT = NP // NS
EPT = E // NS
BLK = 1000
LBLK = 1024


def _mesh():
    return plsc.VectorSubcoreMesh(
        core_axis_name="c", subcore_axis_name="s", num_cores=NC, num_subcores=NS
    )


DH = D // 2


def _mlp_body(z_ref, w1_ref, b1_ref, w2_ref, b2_ref, out0_ref, out1_ref):
    h = jnp.maximum(
        jnp.dot(z_ref[...], w1_ref[...], preferred_element_type=jnp.float32)
        + b1_ref[...],
        0.0,
    )
    m = jnp.dot(h, w2_ref[...], preferred_element_type=jnp.float32) + b2_ref[...]
    out0_ref[...] = m[:, :DH]
    out1_ref[...] = m[:, DH:]


def _mlp(z, W1, b1, W2, b2):
    return pl.pallas_call(
        _mlp_body,
        grid=(N // BLK,),
        in_specs=[
            pl.BlockSpec((BLK, D), lambda i: (i, 0)),
            pl.BlockSpec((D, D), lambda i: (0, 0)),
            pl.BlockSpec((1, D), lambda i: (0, 0)),
            pl.BlockSpec((D, D), lambda i: (0, 0)),
            pl.BlockSpec((1, D), lambda i: (0, 0)),
        ],
        out_specs=[
            pl.BlockSpec((BLK, DH), lambda i: (i, 0)),
            pl.BlockSpec((BLK, DH), lambda i: (i, 0)),
        ],
        out_shape=[
            jax.ShapeDtypeStruct((N, DH), jnp.float32),
            jax.ShapeDtypeStruct((N, DH), jnp.float32),
        ],
    )(z, W1, b1, W2, b2)


def _deg_body(
    src_hbm, alpha_hbm, rdeg_hbm,
    ldeg_v, srch_v, loc_v, rdegf_v, alpha_v, hist_sh, rdeg_sh,
):
    cid = lax.axis_index("c")
    sid = lax.axis_index("s")
    wid = sid * NC + cid

    @pl.loop(0, NP // 16)
    def _z(i):
        ldeg_v[pl.ds(i * 16, 16)] = jnp.zeros((16,), jnp.float32)

    pltpu.sync_copy(src_hbm.at[pl.ds(sid * EPT, EPT)], srch_v)
    ones16 = jnp.full((16,), 1.0, jnp.float32)

    @pl.loop(0, EPT // 16)
    def _h(k):
        idx = srch_v[pl.ds(k * 16, 16)]
        plsc.addupdate_scatter(ldeg_v, [idx], ones16)

    pltpu.sync_copy(ldeg_v, hist_sh.at[sid])
    plsc.subcore_barrier()
    pltpu.sync_copy(hist_sh.at[:, pl.ds(sid * RPT, RPT)], loc_v)

    @pl.loop(0, RPT // 16)
    def _r(c):
        o = pl.ds(c * 16, 16)
        acc = loc_v[0, o]
        for r in range(1, NS):
            acc = acc + loc_v[r, o]
        ldeg_v[o] = 1.0 / (acc + 1e-12)

    pltpu.sync_copy(ldeg_v.at[pl.ds(0, RPT)], rdeg_sh.at[pl.ds(sid * RPT, RPT)])
    plsc.subcore_barrier()

    pltpu.sync_copy(rdeg_sh, rdegf_v)

    @pl.when(cid == 0)
    def _():
        pltpu.sync_copy(rdegf_v.at[pl.ds(sid * RPT, RPT)], rdeg_hbm.at[pl.ds(sid * RPT, RPT)])

    pltpu.sync_copy(src_hbm.at[pl.ds(wid * EPW, EPW)], srch_v.at[pl.ds(0, EPW)])

    @pl.loop(0, EPW // 16)
    def _g(j):
        o = pl.ds(j * 16, 16)
        alpha_v[o] = plsc.load_gather(rdegf_v, [srch_v[o]])

    pltpu.sync_copy(alpha_v, alpha_hbm.at[pl.ds(wid * EPW, EPW)])


def _deg_alpha(src):
    f = pl.kernel(
        _deg_body,
        out_type=(
            jax.ShapeDtypeStruct((E,), jnp.float32),
            jax.ShapeDtypeStruct((NP,), jnp.float32),
        ),
        mesh=_mesh(),
        scratch_types=[
            pltpu.VMEM((NP,), jnp.float32),
            pltpu.VMEM((EPT,), jnp.int32),
            pltpu.VMEM((NS, RPT), jnp.float32),
            pltpu.VMEM((NP,), jnp.float32),
            pltpu.VMEM((EPW,), jnp.float32),
            pltpu.VMEM_SHARED((NS, NP), jnp.float32),
            pltpu.VMEM_SHARED((NP,), jnp.float32),
        ],
        compiler_params=pltpu.CompilerParams(needs_layout_passes=False),
    )
    return f(src)


NCH2 = EPT // CHUNK


def _scatter_body(
    m0_hbm, m1_hbm, src_hbm, dst_hbm, znd_hbm,
    agg_out,
    src_v, dst_v, rows0_v, rows1_v, agg_sh, gs0, gs1, ss0, ss1,
):
    cid = lax.axis_index("c")
    sid = lax.axis_index("s")

    pltpu.sync_copy(src_hbm.at[sid], src_v)
    pltpu.sync_copy(dst_hbm.at[sid], dst_v)
    pltpu.sync_copy(znd_hbm.at[pl.ds(sid * RPT, RPT)], agg_sh.at[pl.ds(sid * RPT, RPT)])
    plsc.subcore_barrier()

    def _run(m_hbm):
        def g_start(j, buf, gs):
            pltpu.async_copy(m_hbm.at[dst_v.at[j]], buf, gs)

        def g_wait(j, buf, gs):
            pltpu.make_async_copy(m_hbm.at[dst_v.at[j]], buf, gs).wait()

        def s_start(j, buf, ss):
            pltpu.async_copy(buf, agg_sh.at[src_v.at[j]], ss, add=True)

        def s_wait(j, buf, ss):
            pltpu.make_async_copy(buf, agg_sh.at[src_v.at[j]], ss).wait()

        g_start(0, rows0_v, gs0)
        g_start(1, rows1_v, gs1)

        @pl.loop(0, NCH2 - 4, step=2)
        def _chunk(j):
            g_wait(j, rows0_v, gs0)
            s_start(j, rows0_v, ss0)
            g_wait(j + 1, rows1_v, gs1)
            s_start(j + 1, rows1_v, ss1)
            s_wait(j, rows0_v, ss0)
            g_start(j + 2, rows0_v, gs0)
            s_wait(j + 1, rows1_v, ss1)
            g_start(j + 3, rows1_v, gs1)

        ja = NCH2 - 4
        g_wait(ja, rows0_v, gs0)
        s_start(ja, rows0_v, ss0)
        g_wait(ja + 1, rows1_v, gs1)
        s_start(ja + 1, rows1_v, ss1)
        s_wait(ja, rows0_v, ss0)
        g_start(ja + 2, rows0_v, gs0)
        s_wait(ja + 1, rows1_v, ss1)
        g_start(ja + 3, rows1_v, gs1)
        g_wait(ja + 2, rows0_v, gs0)
        s_start(ja + 2, rows0_v, ss0)
        g_wait(ja + 3, rows1_v, gs1)
        s_start(ja + 3, rows1_v, ss1)
        s_wait(ja + 2, rows0_v, ss0)
        s_wait(ja + 3, rows1_v, ss1)

    @pl.when(cid == 0)
    def _():
        _run(m0_hbm)

    @pl.when(cid == 1)
    def _():
        _run(m1_hbm)

    plsc.subcore_barrier()
    pltpu.sync_copy(
        agg_sh.at[pl.ds(sid * RPT, RPT)], agg_out.at[cid, pl.ds(sid * RPT, RPT)]
    )


def _scatter(m0, m1, src2, dst2, zeros_nd):
    f = pl.kernel(
        _scatter_body,
        out_type=jax.ShapeDtypeStruct((NC, NP, DH), jnp.float32),
        mesh=_mesh(),
        scratch_types=[
            pltpu.VMEM((NCH2, CHUNK), jnp.int32),
            pltpu.VMEM((NCH2, CHUNK), jnp.int32),
            pltpu.VMEM((CHUNK, DH), jnp.float32),
            pltpu.VMEM((CHUNK, DH), jnp.float32),
            pltpu.VMEM_SHARED((NP, DH), jnp.float32),
            pltpu.SemaphoreType.DMA,
            pltpu.SemaphoreType.DMA,
            pltpu.SemaphoreType.DMA,
            pltpu.SemaphoreType.DMA,
        ],
        compiler_params=pltpu.CompilerParams(use_tc_tiling_on_sc=False),
    )
    return f(m0, m1, src2, dst2, zeros_nd)


def _ln_body(z_ref, a0_ref, a1_ref, rd_ref, g_ref, b_ref, out_ref):
    agg = jnp.concatenate([a0_ref[...], a1_ref[...]], axis=1)
    x = z_ref[...] + agg * rd_ref[...]
    mean = jnp.mean(x, axis=1, keepdims=True)
    xc = x - mean
    var = jnp.mean(xc * xc, axis=1, keepdims=True)
    out_ref[...] = xc * lax.rsqrt(var + 1e-5) * g_ref[...] + b_ref[...]


def _ln(z, agg_p, rdeg, gamma, beta):
    a0, a1 = agg_p[0], agg_p[1]
    rd = rdeg.reshape(NP, 1)
    return pl.pallas_call(
        _ln_body,
        grid=(N // BLK,),
        in_specs=[
            pl.BlockSpec((BLK, D), lambda i: (i, 0)),
            pl.BlockSpec((BLK, DH), lambda i: (i, 0)),
            pl.BlockSpec((BLK, DH), lambda i: (i, 0)),
            pl.BlockSpec((BLK, 1), lambda i: (i, 0)),
            pl.BlockSpec((1, D), lambda i: (0, 0)),
            pl.BlockSpec((1, D), lambda i: (0, 0)),
        ],
        out_specs=pl.BlockSpec((BLK, D), lambda i: (i, 0)),
        out_shape=jax.ShapeDtypeStruct((N, D), jnp.float32),
    )(z, a0, a1, rd, gamma, beta)


def kernel(z, edge_index, W1, b1, W2, b2, gamma, beta):
    ei = edge_index.astype(jnp.int32)
    src = ei[0]
    dst = ei[1]
    src2 = src.reshape(NS, NCH2, CHUNK)
    dst2 = dst.reshape(NS, NCH2, CHUNK)
    zeros_nd = jnp.zeros((NP, DH), jnp.float32)

    m0, m1 = _mlp(z, W1, b1.reshape(1, D), W2, b2.reshape(1, D))
    alpha, rdeg = _deg_alpha(src)
    agg_p = _scatter(m0, m1, src2, dst2, zeros_nd)
    out = _ln(z, agg_p, rdeg, gamma.reshape(1, D), beta.reshape(1, D))
    return (out, alpha)


# speedup vs baseline: 12.9673x; 1.0002x over previous
import jax
import jax.numpy as jnp
from jax import lax
from jax.experimental import pallas as pl
from jax.experimental.pallas import tpu as pltpu
from jax.experimental.pallas import tpu_sc as plsc

D = 128
N = 10000
E = 320000
NP = 10240
NC = 2
NS = 16
NW = NC * NS
EPW = E // NW
CHUNK = 80
NCHUNK = EPW // CHUNK
RPT = NP // NS
EPT = E // NS
BLK = 1000
LBLK = 1024


def _mesh():
    return plsc.VectorSubcoreMesh(
        core_axis_name="c", subcore_axis_name="s", num_cores=NC, num_subcores=NS
    )


DH = D // 2


def _mlp_body(z_ref, w1_ref, b1_ref, w2_ref, b2_ref, out0_ref, out1_ref):
    h = jnp.maximum(
        jnp.dot(z_ref[...], w1_ref[...], preferred_element_type=jnp.float32)
        + b1_ref[...],
        0.0,
    )
    m = jnp.dot(h, w2_ref[...], preferred_element_type=jnp.float32) + b2_ref[...]
    out0_ref[...] = m[:, :DH]
    out1_ref[...] = m[:, DH:]


def _mlp(z, W1, b1, W2, b2):
    return pl.pallas_call(
        _mlp_body,
        grid=(N // BLK,),
        in_specs=[
            pl.BlockSpec((BLK, D), lambda i: (i, 0)),
            pl.BlockSpec((D, D), lambda i: (0, 0)),
            pl.BlockSpec((1, D), lambda i: (0, 0)),
            pl.BlockSpec((D, D), lambda i: (0, 0)),
            pl.BlockSpec((1, D), lambda i: (0, 0)),
        ],
        out_specs=[
            pl.BlockSpec((BLK, DH), lambda i: (i, 0)),
            pl.BlockSpec((BLK, DH), lambda i: (i, 0)),
        ],
        out_shape=[
            jax.ShapeDtypeStruct((N, DH), jnp.float32),
            jax.ShapeDtypeStruct((N, DH), jnp.float32),
        ],
    )(z, W1, b1, W2, b2)


NCH2 = EPT // CHUNK


def _scatter_body(
    m0_hbm, m1_hbm, src_hbm, dst_hbm, znd_hbm, zn_hbm,
    agg_out, alpha_hbm, rdeg_hbm,
    src_v, dst_v, rows0_v, rows1_v, ones_v, ldeg_v, alpha_v,
    agg_sh, deg_sh, gs0, gs1, ss0, ss1, ds0,
):
    cid = lax.axis_index("c")
    sid = lax.axis_index("s")
    wid = sid * NC + cid

    pltpu.sync_copy(src_hbm.at[sid], src_v)
    pltpu.sync_copy(dst_hbm.at[sid], dst_v)
    pltpu.sync_copy(znd_hbm.at[pl.ds(sid * RPT, RPT)], agg_sh.at[pl.ds(sid * RPT, RPT)])
    pltpu.sync_copy(zn_hbm.at[pl.ds(sid * RPT, RPT)], deg_sh.at[pl.ds(sid * RPT, RPT)])
    for i in range(CHUNK // 16):
        ones_v[pl.ds(i * 16, 16)] = jnp.full((16,), 1.0, jnp.float32)
    plsc.subcore_barrier()

    def d_start(j):
        pltpu.async_copy(ones_v, deg_sh.at[src_v.at[j]], ds0, add=True)

    def _run(m_hbm):
        def g_start(j, buf, gs):
            pltpu.async_copy(m_hbm.at[dst_v.at[j]], buf, gs)

        def g_wait(j, buf, gs):
            pltpu.make_async_copy(m_hbm.at[dst_v.at[j]], buf, gs).wait()

        def s_start(j, buf, ss):
            pltpu.async_copy(buf, agg_sh.at[src_v.at[j]], ss, add=True)

        def s_wait(j, buf, ss):
            pltpu.make_async_copy(buf, agg_sh.at[src_v.at[j]], ss).wait()

        g_start(0, rows0_v, gs0)
        g_start(1, rows1_v, gs1)

        @pl.loop(0, NCH2 - 4, step=2)
        def _chunk(j):
            g_wait(j, rows0_v, gs0)
            s_start(j, rows0_v, ss0)
            g_wait(j + 1, rows1_v, gs1)
            s_start(j + 1, rows1_v, ss1)
            d_start(j)
            d_start(j + 1)
            s_wait(j, rows0_v, ss0)
            g_start(j + 2, rows0_v, gs0)
            s_wait(j + 1, rows1_v, ss1)
            g_start(j + 3, rows1_v, gs1)

        ja = NCH2 - 4
        g_wait(ja, rows0_v, gs0)
        s_start(ja, rows0_v, ss0)
        g_wait(ja + 1, rows1_v, gs1)
        s_start(ja + 1, rows1_v, ss1)
        s_wait(ja, rows0_v, ss0)
        g_start(ja + 2, rows0_v, gs0)
        s_wait(ja + 1, rows1_v, ss1)
        g_start(ja + 3, rows1_v, gs1)
        g_wait(ja + 2, rows0_v, gs0)
        s_start(ja + 2, rows0_v, ss0)
        g_wait(ja + 3, rows1_v, gs1)
        s_start(ja + 3, rows1_v, ss1)
        d_start(ja)
        d_start(ja + 1)
        d_start(ja + 2)
        d_start(ja + 3)
        s_wait(ja + 2, rows0_v, ss0)
        s_wait(ja + 3, rows1_v, ss1)

        @pl.loop(0, NCH2)
        def _dd(j):
            pltpu.make_async_copy(ones_v, deg_sh.at[src_v.at[j]], ds0).wait()

    @pl.when(cid == 0)
    def _():
        _run(m0_hbm)

    @pl.when(cid == 1)
    def _():
        _run(m1_hbm)

    plsc.subcore_barrier()
    pltpu.sync_copy(
        agg_sh.at[pl.ds(sid * RPT, RPT)], agg_out.at[cid, pl.ds(sid * RPT, RPT)]
    )

    pltpu.sync_copy(deg_sh, ldeg_v)

    @pl.loop(0, NP // 16)
    def _r(c):
        o = pl.ds(c * 16, 16)
        ldeg_v[o] = 1.0 / (ldeg_v[o] + 1e-12)

    @pl.when(cid == 0)
    def _():
        pltpu.sync_copy(ldeg_v.at[pl.ds(sid * RPT, RPT)], rdeg_hbm.at[pl.ds(sid * RPT, RPT)])

    @pl.loop(0, NCHUNK)
    def _g(j2):
        for k in range(CHUNK // 16):
            o = pl.ds(k * 16, 16)
            idx = src_v[cid * NCHUNK + j2, o]
            alpha_v[j2, o] = plsc.load_gather(ldeg_v, [idx])

    pltpu.sync_copy(alpha_v, alpha_hbm.at[wid])


def _scatter(m0, m1, src2, dst2, zeros_nd, zeros_n):
    f = pl.kernel(
        _scatter_body,
        out_type=(
            jax.ShapeDtypeStruct((NC, NP, DH), jnp.float32),
            jax.ShapeDtypeStruct((NW, NCHUNK, CHUNK), jnp.float32),
            jax.ShapeDtypeStruct((NP,), jnp.float32),
        ),
        mesh=_mesh(),
        scratch_types=[
            pltpu.VMEM((NCH2, CHUNK), jnp.int32),
            pltpu.VMEM((NCH2, CHUNK), jnp.int32),
            pltpu.VMEM((CHUNK, DH), jnp.float32),
            pltpu.VMEM((CHUNK, DH), jnp.float32),
            pltpu.VMEM((CHUNK,), jnp.float32),
            pltpu.VMEM((NP,), jnp.float32),
            pltpu.VMEM((NCHUNK, CHUNK), jnp.float32),
            pltpu.VMEM_SHARED((NP, DH), jnp.float32),
            pltpu.VMEM_SHARED((NP,), jnp.float32),
            pltpu.SemaphoreType.DMA,
            pltpu.SemaphoreType.DMA,
            pltpu.SemaphoreType.DMA,
            pltpu.SemaphoreType.DMA,
            pltpu.SemaphoreType.DMA,
        ],
        compiler_params=pltpu.CompilerParams(
            use_tc_tiling_on_sc=False, needs_layout_passes=False
        ),
    )
    return f(m0, m1, src2, dst2, zeros_nd, zeros_n)


def _ln_body(z_ref, a0_ref, a1_ref, rd_ref, g_ref, b_ref, out_ref):
    agg = jnp.concatenate([a0_ref[...], a1_ref[...]], axis=1)
    x = z_ref[...] + agg * rd_ref[...]
    mean = jnp.mean(x, axis=1, keepdims=True)
    xc = x - mean
    var = jnp.mean(xc * xc, axis=1, keepdims=True)
    out_ref[...] = xc * lax.rsqrt(var + 1e-5) * g_ref[...] + b_ref[...]


def _ln(z, agg_p, rdeg, gamma, beta):
    a0, a1 = agg_p[0], agg_p[1]
    rd = rdeg.reshape(NP, 1)
    return pl.pallas_call(
        _ln_body,
        grid=(N // BLK,),
        in_specs=[
            pl.BlockSpec((BLK, D), lambda i: (i, 0)),
            pl.BlockSpec((BLK, DH), lambda i: (i, 0)),
            pl.BlockSpec((BLK, DH), lambda i: (i, 0)),
            pl.BlockSpec((BLK, 1), lambda i: (i, 0)),
            pl.BlockSpec((1, D), lambda i: (0, 0)),
            pl.BlockSpec((1, D), lambda i: (0, 0)),
        ],
        out_specs=pl.BlockSpec((BLK, D), lambda i: (i, 0)),
        out_shape=jax.ShapeDtypeStruct((N, D), jnp.float32),
    )(z, a0, a1, rd, gamma, beta)


def kernel(z, edge_index, W1, b1, W2, b2, gamma, beta):
    ei = edge_index.astype(jnp.int32)
    src = ei[0]
    dst = ei[1]
    src2 = src.reshape(NS, NCH2, CHUNK)
    dst2 = dst.reshape(NS, NCH2, CHUNK)
    zeros_nd = jnp.zeros((NP, DH), jnp.float32)
    zeros_n = jnp.zeros((NP,), jnp.float32)

    m0, m1 = _mlp(z, W1, b1.reshape(1, D), W2, b2.reshape(1, D))
    agg_p, alpha3, rdeg = _scatter(m0, m1, src2, dst2, zeros_nd, zeros_n)
    alpha = alpha3.reshape(E)
    out = _ln(z, agg_p, rdeg, gamma.reshape(1, D), beta.reshape(1, D))
    return (out, alpha)


# speedup vs baseline: 13.8679x; 1.0695x over previous
import jax
import jax.numpy as jnp
from jax import lax
from jax.experimental import pallas as pl
from jax.experimental.pallas import tpu as pltpu
from jax.experimental.pallas import tpu_sc as plsc

D = 128
N = 10000
E = 320000
NP = 10240
NC = 2
NS = 16
NW = NC * NS
EPW = E // NW
CHUNK = 80
NCHUNK = EPW // CHUNK
RPT = NP // NS
EPT = E // NS
BLK = 1000
LBLK = 1024


def _mesh():
    return plsc.VectorSubcoreMesh(
        core_axis_name="c", subcore_axis_name="s", num_cores=NC, num_subcores=NS
    )


DH = D // 2


def _mlp_body(z_ref, w1_ref, b1_ref, w2_ref, b2_ref, out0_ref, out1_ref):
    h = jnp.maximum(
        jnp.dot(z_ref[...], w1_ref[...], preferred_element_type=jnp.float32)
        + b1_ref[...],
        0.0,
    )
    m = jnp.dot(h, w2_ref[...], preferred_element_type=jnp.float32) + b2_ref[...]
    out0_ref[...] = m[:, :DH]
    out1_ref[...] = m[:, DH:]


def _mlp(z, W1, b1, W2, b2):
    return pl.pallas_call(
        _mlp_body,
        grid=(N // BLK,),
        in_specs=[
            pl.BlockSpec((BLK, D), lambda i: (i, 0)),
            pl.BlockSpec((D, D), lambda i: (0, 0)),
            pl.BlockSpec((1, D), lambda i: (0, 0)),
            pl.BlockSpec((D, D), lambda i: (0, 0)),
            pl.BlockSpec((1, D), lambda i: (0, 0)),
        ],
        out_specs=[
            pl.BlockSpec((BLK, DH), lambda i: (i, 0)),
            pl.BlockSpec((BLK, DH), lambda i: (i, 0)),
        ],
        out_shape=[
            jax.ShapeDtypeStruct((N, DH), jnp.float32),
            jax.ShapeDtypeStruct((N, DH), jnp.float32),
        ],
    )(z, W1, b1, W2, b2)


NCH2 = EPT // CHUNK


def _scatter_body(
    e4_hbm, m0_hbm, m1_hbm, znd_hbm, zn_hbm,
    agg_out, alpha_hbm, rdeg_hbm,
    src_v, dst_v, rows0_v, rows1_v, ones_v, ldeg_v, alpha_v,
    agg_sh, deg_sh, gs0, gs1, ss0, ss1, ds0,
):
    cid = lax.axis_index("c")
    sid = lax.axis_index("s")
    wid = sid * NC + cid

    pltpu.sync_copy(e4_hbm.at[0, sid], src_v)
    pltpu.sync_copy(e4_hbm.at[1, sid], dst_v)
    pltpu.sync_copy(znd_hbm.at[pl.ds(sid * RPT, RPT)], agg_sh.at[pl.ds(sid * RPT, RPT)])
    pltpu.sync_copy(zn_hbm.at[pl.ds(sid * RPT, RPT)], deg_sh.at[pl.ds(sid * RPT, RPT)])
    for i in range(CHUNK // 16):
        ones_v[pl.ds(i * 16, 16)] = jnp.full((16,), 1.0, jnp.float32)
    plsc.subcore_barrier()

    def d_start(j):
        pltpu.async_copy(ones_v, deg_sh.at[src_v.at[j]], ds0, add=True)

    def _run(m_hbm):
        def g_start(j, buf, gs):
            pltpu.async_copy(m_hbm.at[dst_v.at[j]], buf, gs)

        def g_wait(j, buf, gs):
            pltpu.make_async_copy(m_hbm.at[dst_v.at[j]], buf, gs).wait()

        def s_start(j, buf, ss):
            pltpu.async_copy(buf, agg_sh.at[src_v.at[j]], ss, add=True)

        def s_wait(j, buf, ss):
            pltpu.make_async_copy(buf, agg_sh.at[src_v.at[j]], ss).wait()

        g_start(0, rows0_v, gs0)
        g_start(1, rows1_v, gs1)

        @pl.loop(0, NCH2 - 4, step=2)
        def _chunk(j):
            g_wait(j, rows0_v, gs0)
            s_start(j, rows0_v, ss0)
            g_wait(j + 1, rows1_v, gs1)
            s_start(j + 1, rows1_v, ss1)
            d_start(j)
            d_start(j + 1)
            s_wait(j, rows0_v, ss0)
            g_start(j + 2, rows0_v, gs0)
            s_wait(j + 1, rows1_v, ss1)
            g_start(j + 3, rows1_v, gs1)

        ja = NCH2 - 4
        g_wait(ja, rows0_v, gs0)
        s_start(ja, rows0_v, ss0)
        g_wait(ja + 1, rows1_v, gs1)
        s_start(ja + 1, rows1_v, ss1)
        s_wait(ja, rows0_v, ss0)
        g_start(ja + 2, rows0_v, gs0)
        s_wait(ja + 1, rows1_v, ss1)
        g_start(ja + 3, rows1_v, gs1)
        g_wait(ja + 2, rows0_v, gs0)
        s_start(ja + 2, rows0_v, ss0)
        g_wait(ja + 3, rows1_v, gs1)
        s_start(ja + 3, rows1_v, ss1)
        d_start(ja)
        d_start(ja + 1)
        d_start(ja + 2)
        d_start(ja + 3)
        s_wait(ja + 2, rows0_v, ss0)
        s_wait(ja + 3, rows1_v, ss1)

        @pl.loop(0, NCH2)
        def _dd(j):
            pltpu.make_async_copy(ones_v, deg_sh.at[src_v.at[j]], ds0).wait()

    @pl.when(cid == 0)
    def _():
        _run(m0_hbm)

    @pl.when(cid == 1)
    def _():
        _run(m1_hbm)

    plsc.subcore_barrier()
    pltpu.sync_copy(
        agg_sh.at[pl.ds(sid * RPT, RPT)], agg_out.at[cid, pl.ds(sid * RPT, RPT)]
    )

    pltpu.sync_copy(deg_sh, ldeg_v)

    @pl.loop(0, NP // 16)
    def _r(c):
        o = pl.ds(c * 16, 16)
        ldeg_v[o] = 1.0 / (ldeg_v[o] + 1e-12)

    @pl.when(cid == 0)
    def _():
        pltpu.sync_copy(ldeg_v.at[pl.ds(sid * RPT, RPT)], rdeg_hbm.at[pl.ds(sid * RPT, RPT)])

    @pl.loop(0, NCHUNK)
    def _g(j2):
        for k in range(CHUNK // 16):
            idx = src_v[cid * NCHUNK + j2, pl.ds(k * 16, 16)]
            alpha_v[pl.ds(j2 * CHUNK + k * 16, 16)] = plsc.load_gather(ldeg_v, [idx])

    pltpu.sync_copy(alpha_v, alpha_hbm.at[pl.ds(wid * EPW, EPW)])


def _scatter(e4, m0, m1, zeros_nd, zeros_n):
    f = pl.kernel(
        _scatter_body,
        out_type=(
            jax.ShapeDtypeStruct((NC, NP, DH), jnp.float32),
            jax.ShapeDtypeStruct((E,), jnp.float32),
            jax.ShapeDtypeStruct((NP,), jnp.float32),
        ),
        mesh=_mesh(),
        scratch_types=[
            pltpu.VMEM((NCH2, CHUNK), jnp.int32),
            pltpu.VMEM((NCH2, CHUNK), jnp.int32),
            pltpu.VMEM((CHUNK, DH), jnp.float32),
            pltpu.VMEM((CHUNK, DH), jnp.float32),
            pltpu.VMEM((CHUNK,), jnp.float32),
            pltpu.VMEM((NP,), jnp.float32),
            pltpu.VMEM((EPW,), jnp.float32),
            pltpu.VMEM_SHARED((NP, DH), jnp.float32),
            pltpu.VMEM_SHARED((NP,), jnp.float32),
            pltpu.SemaphoreType.DMA,
            pltpu.SemaphoreType.DMA,
            pltpu.SemaphoreType.DMA,
            pltpu.SemaphoreType.DMA,
            pltpu.SemaphoreType.DMA,
        ],
        compiler_params=pltpu.CompilerParams(
            use_tc_tiling_on_sc=False, needs_layout_passes=False
        ),
    )
    return f(e4, m0, m1, zeros_nd, zeros_n)


def _ln_body(z_ref, agg_ref, rd_ref, g_ref, b_ref, out_ref):
    agg = jnp.concatenate([agg_ref[0], agg_ref[1]], axis=1)
    x = z_ref[...] + agg * rd_ref[...]
    mean = jnp.mean(x, axis=1, keepdims=True)
    xc = x - mean
    var = jnp.mean(xc * xc, axis=1, keepdims=True)
    out_ref[...] = xc * lax.rsqrt(var + 1e-5) * g_ref[...] + b_ref[...]


def _ln(z, agg_p, rdeg, gamma, beta):
    rd = rdeg.reshape(NP, 1)
    return pl.pallas_call(
        _ln_body,
        grid=(N // BLK,),
        in_specs=[
            pl.BlockSpec((BLK, D), lambda i: (i, 0)),
            pl.BlockSpec((NC, BLK, DH), lambda i: (0, i, 0)),
            pl.BlockSpec((BLK, 1), lambda i: (i, 0)),
            pl.BlockSpec((1, D), lambda i: (0, 0)),
            pl.BlockSpec((1, D), lambda i: (0, 0)),
        ],
        out_specs=pl.BlockSpec((BLK, D), lambda i: (i, 0)),
        out_shape=jax.ShapeDtypeStruct((N, D), jnp.float32),
    )(z, agg_p, rd, gamma, beta)


def kernel(z, edge_index, W1, b1, W2, b2, gamma, beta):
    e4 = edge_index.astype(jnp.int32).reshape(2, NS, NCH2, CHUNK)
    zeros_nd = jnp.zeros((NP, DH), jnp.float32)
    zeros_n = jnp.zeros((NP,), jnp.float32)

    m0, m1 = _mlp(z, W1, b1.reshape(1, D), W2, b2.reshape(1, D))
    agg_p, alpha, rdeg = _scatter(e4, m0, m1, zeros_nd, zeros_n)
    out = _ln(z, agg_p, rdeg, gamma.reshape(1, D), beta.reshape(1, D))
    return (out, alpha)


# speedup vs baseline: 14.1946x; 1.0236x over previous
import jax
import jax.numpy as jnp
from jax import lax
from jax.experimental import pallas as pl
from jax.experimental.pallas import tpu as pltpu
from jax.experimental.pallas import tpu_sc as plsc

D = 128
N = 10000
E = 320000
NP = 10240
NC = 2
NS = 16
NW = NC * NS
EPW = E // NW
CHUNK = 80
NCHUNK = EPW // CHUNK
RPT = NP // NS
EPT = E // NS
BLK = 2000
LBLK = 1024


def _mesh():
    return plsc.VectorSubcoreMesh(
        core_axis_name="c", subcore_axis_name="s", num_cores=NC, num_subcores=NS
    )


DH = D // 2


def _mlp_body(z_ref, w1_ref, b1_ref, w2_ref, b2_ref, out0_ref, out1_ref):
    h = jnp.maximum(
        jnp.dot(z_ref[...], w1_ref[...], preferred_element_type=jnp.float32)
        + b1_ref[...],
        0.0,
    )
    m = jnp.dot(h, w2_ref[...], preferred_element_type=jnp.float32) + b2_ref[...]
    out0_ref[...] = m[:, :DH]
    out1_ref[...] = m[:, DH:]


def _mlp(z, W1, b1, W2, b2):
    return pl.pallas_call(
        _mlp_body,
        grid=(N // BLK,),
        in_specs=[
            pl.BlockSpec((BLK, D), lambda i: (i, 0)),
            pl.BlockSpec((D, D), lambda i: (0, 0)),
            pl.BlockSpec((1, D), lambda i: (0, 0)),
            pl.BlockSpec((D, D), lambda i: (0, 0)),
            pl.BlockSpec((1, D), lambda i: (0, 0)),
        ],
        out_specs=[
            pl.BlockSpec((BLK, DH), lambda i: (i, 0)),
            pl.BlockSpec((BLK, DH), lambda i: (i, 0)),
        ],
        out_shape=[
            jax.ShapeDtypeStruct((N, DH), jnp.float32),
            jax.ShapeDtypeStruct((N, DH), jnp.float32),
        ],
    )(z, W1, b1, W2, b2)


NCH2 = EPT // CHUNK


def _scatter_body(
    e4_hbm, m0_hbm, m1_hbm, znd_hbm, zn_hbm,
    agg_out, alpha_hbm, rdeg_hbm,
    src_v, dst_v, rows0_v, rows1_v, ones_v, ldeg_v, alpha_v,
    agg_sh, deg_sh, gs0, gs1, ss0, ss1, ds0,
):
    cid = lax.axis_index("c")
    sid = lax.axis_index("s")
    wid = sid * NC + cid

    pltpu.sync_copy(e4_hbm.at[0, sid], src_v)
    pltpu.sync_copy(e4_hbm.at[1, sid], dst_v)
    pltpu.sync_copy(znd_hbm.at[pl.ds(sid * RPT, RPT)], agg_sh.at[pl.ds(sid * RPT, RPT)])
    pltpu.sync_copy(zn_hbm.at[pl.ds(sid * RPT, RPT)], deg_sh.at[pl.ds(sid * RPT, RPT)])
    for i in range(CHUNK // 16):
        ones_v[pl.ds(i * 16, 16)] = jnp.full((16,), 1.0, jnp.float32)
    plsc.subcore_barrier()

    def d_start(j):
        pltpu.async_copy(ones_v, deg_sh.at[src_v.at[j]], ds0, add=True)

    def _run(m_hbm):
        def g_start(j, buf, gs):
            pltpu.async_copy(m_hbm.at[dst_v.at[j]], buf, gs)

        def g_wait(j, buf, gs):
            pltpu.make_async_copy(m_hbm.at[dst_v.at[j]], buf, gs).wait()

        def s_start(j, buf, ss):
            pltpu.async_copy(buf, agg_sh.at[src_v.at[j]], ss, add=True)

        def s_wait(j, buf, ss):
            pltpu.make_async_copy(buf, agg_sh.at[src_v.at[j]], ss).wait()

        g_start(0, rows0_v, gs0)
        g_start(1, rows1_v, gs1)

        @pl.loop(0, NCH2 - 4, step=2)
        def _chunk(j):
            g_wait(j, rows0_v, gs0)
            s_start(j, rows0_v, ss0)
            g_wait(j + 1, rows1_v, gs1)
            s_start(j + 1, rows1_v, ss1)
            d_start(j)
            d_start(j + 1)
            s_wait(j, rows0_v, ss0)
            g_start(j + 2, rows0_v, gs0)
            s_wait(j + 1, rows1_v, ss1)
            g_start(j + 3, rows1_v, gs1)

        ja = NCH2 - 4
        g_wait(ja, rows0_v, gs0)
        s_start(ja, rows0_v, ss0)
        g_wait(ja + 1, rows1_v, gs1)
        s_start(ja + 1, rows1_v, ss1)
        s_wait(ja, rows0_v, ss0)
        g_start(ja + 2, rows0_v, gs0)
        s_wait(ja + 1, rows1_v, ss1)
        g_start(ja + 3, rows1_v, gs1)
        g_wait(ja + 2, rows0_v, gs0)
        s_start(ja + 2, rows0_v, ss0)
        g_wait(ja + 3, rows1_v, gs1)
        s_start(ja + 3, rows1_v, ss1)
        d_start(ja)
        d_start(ja + 1)
        d_start(ja + 2)
        d_start(ja + 3)
        s_wait(ja + 2, rows0_v, ss0)
        s_wait(ja + 3, rows1_v, ss1)

        pltpu.make_async_copy(e4_hbm.at[0, sid], src_v, ds0).wait()

    @pl.when(cid == 0)
    def _():
        _run(m0_hbm)

    @pl.when(cid == 1)
    def _():
        _run(m1_hbm)

    plsc.subcore_barrier()
    pltpu.sync_copy(
        agg_sh.at[pl.ds(sid * RPT, RPT)], agg_out.at[cid, pl.ds(sid * RPT, RPT)]
    )

    pltpu.sync_copy(deg_sh, ldeg_v)

    @pl.loop(0, NP // 16)
    def _r(c):
        o = pl.ds(c * 16, 16)
        ldeg_v[o] = 1.0 / (ldeg_v[o] + 1e-12)

    @pl.when(cid == 0)
    def _():
        pltpu.sync_copy(ldeg_v.at[pl.ds(sid * RPT, RPT)], rdeg_hbm.at[pl.ds(sid * RPT, RPT)])

    @pl.loop(0, NCHUNK)
    def _g(j2):
        for k in range(CHUNK // 16):
            idx = src_v[cid * NCHUNK + j2, pl.ds(k * 16, 16)]
            alpha_v[pl.ds(j2 * CHUNK + k * 16, 16)] = plsc.load_gather(ldeg_v, [idx])

    pltpu.sync_copy(alpha_v, alpha_hbm.at[pl.ds(wid * EPW, EPW)])


def _scatter(e4, m0, m1, zeros_nd, zeros_n):
    f = pl.kernel(
        _scatter_body,
        out_type=(
            jax.ShapeDtypeStruct((NC, NP, DH), jnp.float32),
            jax.ShapeDtypeStruct((E,), jnp.float32),
            jax.ShapeDtypeStruct((NP,), jnp.float32),
        ),
        mesh=_mesh(),
        scratch_types=[
            pltpu.VMEM((NCH2, CHUNK), jnp.int32),
            pltpu.VMEM((NCH2, CHUNK), jnp.int32),
            pltpu.VMEM((CHUNK, DH), jnp.float32),
            pltpu.VMEM((CHUNK, DH), jnp.float32),
            pltpu.VMEM((CHUNK,), jnp.float32),
            pltpu.VMEM((NP,), jnp.float32),
            pltpu.VMEM((EPW,), jnp.float32),
            pltpu.VMEM_SHARED((NP, DH), jnp.float32),
            pltpu.VMEM_SHARED((NP,), jnp.float32),
            pltpu.SemaphoreType.DMA,
            pltpu.SemaphoreType.DMA,
            pltpu.SemaphoreType.DMA,
            pltpu.SemaphoreType.DMA,
            pltpu.SemaphoreType.DMA,
        ],
        compiler_params=pltpu.CompilerParams(
            use_tc_tiling_on_sc=False, needs_layout_passes=False
        ),
    )
    return f(e4, m0, m1, zeros_nd, zeros_n)


def _ln_body(z_ref, agg_ref, rd_ref, g_ref, b_ref, out_ref):
    agg = jnp.concatenate([agg_ref[0], agg_ref[1]], axis=1)
    x = z_ref[...] + agg * rd_ref[...]
    mean = jnp.mean(x, axis=1, keepdims=True)
    xc = x - mean
    var = jnp.mean(xc * xc, axis=1, keepdims=True)
    out_ref[...] = xc * lax.rsqrt(var + 1e-5) * g_ref[...] + b_ref[...]


def _ln(z, agg_p, rdeg, gamma, beta):
    rd = rdeg.reshape(NP, 1)
    return pl.pallas_call(
        _ln_body,
        grid=(N // BLK,),
        in_specs=[
            pl.BlockSpec((BLK, D), lambda i: (i, 0)),
            pl.BlockSpec((NC, BLK, DH), lambda i: (0, i, 0)),
            pl.BlockSpec((BLK, 1), lambda i: (i, 0)),
            pl.BlockSpec((1, D), lambda i: (0, 0)),
            pl.BlockSpec((1, D), lambda i: (0, 0)),
        ],
        out_specs=pl.BlockSpec((BLK, D), lambda i: (i, 0)),
        out_shape=jax.ShapeDtypeStruct((N, D), jnp.float32),
    )(z, agg_p, rd, gamma, beta)


def kernel(z, edge_index, W1, b1, W2, b2, gamma, beta):
    e4 = edge_index.astype(jnp.int32).reshape(2, NS, NCH2, CHUNK)
    zeros_nd = jnp.zeros((NP, DH), jnp.float32)
    zeros_n = jnp.zeros((NP,), jnp.float32)

    m0, m1 = _mlp(z, W1, b1.reshape(1, D), W2, b2.reshape(1, D))
    agg_p, alpha, rdeg = _scatter(e4, m0, m1, zeros_nd, zeros_n)
    out = _ln(z, agg_p, rdeg, gamma.reshape(1, D), beta.reshape(1, D))
    return (out, alpha)


# speedup vs baseline: 14.8142x; 1.0437x over previous
import jax
import jax.numpy as jnp
from jax import lax
from jax.experimental import pallas as pl
from jax.experimental.pallas import tpu as pltpu
from jax.experimental.pallas import tpu_sc as plsc

D = 128
N = 10000
E = 320000
NP = 10240
NC = 2
NS = 16
NW = NC * NS
EPW = E // NW
CHUNK = 80
NCHUNK = EPW // CHUNK
RPT = NP // NS
EPT = E // NS
BLK = 2000
LBLK = 1024


def _mesh():
    return plsc.VectorSubcoreMesh(
        core_axis_name="c", subcore_axis_name="s", num_cores=NC, num_subcores=NS
    )


DH = D // 2


def _mlp_body(z_ref, w1_ref, b1_ref, w2_ref, b2_ref, out0_ref, out1_ref):
    h = jnp.maximum(
        jnp.dot(z_ref[...], w1_ref[...], preferred_element_type=jnp.float32)
        + b1_ref[...],
        0.0,
    )
    m = jnp.dot(h, w2_ref[...], preferred_element_type=jnp.float32) + b2_ref[...]
    out0_ref[...] = m[:, :DH]
    out1_ref[...] = m[:, DH:]


def _mlp(z, W1, b1, W2, b2):
    return pl.pallas_call(
        _mlp_body,
        grid=(N // BLK,),
        in_specs=[
            pl.BlockSpec((BLK, D), lambda i: (i, 0)),
            pl.BlockSpec((D, D), lambda i: (0, 0)),
            pl.BlockSpec((1, D), lambda i: (0, 0)),
            pl.BlockSpec((D, D), lambda i: (0, 0)),
            pl.BlockSpec((1, D), lambda i: (0, 0)),
        ],
        out_specs=[
            pl.BlockSpec((BLK, DH), lambda i: (i, 0)),
            pl.BlockSpec((BLK, DH), lambda i: (i, 0)),
        ],
        out_shape=[
            jax.ShapeDtypeStruct((N, DH), jnp.float32),
            jax.ShapeDtypeStruct((N, DH), jnp.float32),
        ],
    )(z, W1, b1, W2, b2)


NCH2 = EPT // CHUNK


def _scatter_body(
    e4_hbm, m0_hbm, m1_hbm, znd_hbm, zn_hbm,
    agg_out, alpha_hbm, rdeg_hbm,
    src_v, dst_v, rows0_v, rows1_v, ones_v, ldeg_v, alpha_v,
    agg_sh, deg_sh, gs0, gs1, ss0, ss1, ds0,
):
    cid = lax.axis_index("c")
    sid = lax.axis_index("s")
    wid = sid * NC + cid

    pltpu.sync_copy(e4_hbm.at[0, sid], src_v)
    pltpu.sync_copy(e4_hbm.at[1, sid], dst_v)
    pltpu.sync_copy(znd_hbm.at[pl.ds(sid * RPT, RPT)], agg_sh.at[pl.ds(sid * RPT, RPT)])
    pltpu.sync_copy(zn_hbm.at[pl.ds(sid * RPT, RPT)], deg_sh.at[pl.ds(sid * RPT, RPT)])
    for i in range(CHUNK // 16):
        ones_v[pl.ds(i * 16, 16)] = jnp.full((16,), 1.0, jnp.float32)
    plsc.subcore_barrier()

    def d_start(j):
        pltpu.async_copy(ones_v, deg_sh.at[src_v.at[j]], ds0, add=True)

    def _run(mh):
        def g_start(j, buf, gs):
            pltpu.async_copy(mh.at[dst_v.at[j]], buf, gs)

        def g_wait(j, buf, gs):
            pltpu.make_async_copy(mh.at[dst_v.at[j]], buf, gs).wait()

        def s_start(j, buf, ss):
            pltpu.async_copy(buf, agg_sh.at[src_v.at[j]], ss, add=True)

        def s_wait(j, buf, ss):
            pltpu.make_async_copy(buf, agg_sh.at[src_v.at[j]], ss).wait()

        g_start(0, rows0_v, gs0)
        g_start(1, rows1_v, gs1)

        @pl.loop(0, NCH2 - 4, step=2)
        def _chunk(j):
            g_wait(j, rows0_v, gs0)
            s_start(j, rows0_v, ss0)
            g_wait(j + 1, rows1_v, gs1)
            s_start(j + 1, rows1_v, ss1)
            d_start(j)
            d_start(j + 1)
            s_wait(j, rows0_v, ss0)
            g_start(j + 2, rows0_v, gs0)
            s_wait(j + 1, rows1_v, ss1)
            g_start(j + 3, rows1_v, gs1)

        ja = NCH2 - 4
        g_wait(ja, rows0_v, gs0)
        s_start(ja, rows0_v, ss0)
        g_wait(ja + 1, rows1_v, gs1)
        s_start(ja + 1, rows1_v, ss1)
        s_wait(ja, rows0_v, ss0)
        g_start(ja + 2, rows0_v, gs0)
        s_wait(ja + 1, rows1_v, ss1)
        g_start(ja + 3, rows1_v, gs1)
        g_wait(ja + 2, rows0_v, gs0)
        s_start(ja + 2, rows0_v, ss0)
        g_wait(ja + 3, rows1_v, gs1)
        s_start(ja + 3, rows1_v, ss1)
        d_start(ja)
        d_start(ja + 1)
        d_start(ja + 2)
        d_start(ja + 3)
        s_wait(ja + 2, rows0_v, ss0)
        s_wait(ja + 3, rows1_v, ss1)

        pltpu.make_async_copy(e4_hbm.at[0, sid], src_v, ds0).wait()

    @pl.when(cid == 0)
    def _():
        _run(m0_hbm)

    @pl.when(cid == 1)
    def _():
        _run(m1_hbm)

    plsc.subcore_barrier()
    pltpu.sync_copy(
        agg_sh.at[pl.ds(sid * RPT, RPT)],
        agg_out.at[pl.ds(sid * RPT, RPT), pl.ds(cid * DH, DH)],
    )

    pltpu.sync_copy(deg_sh, ldeg_v)

    @pl.loop(0, NP // 16)
    def _r(c):
        o = pl.ds(c * 16, 16)
        ldeg_v[o] = 1.0 / (ldeg_v[o] + 1e-12)

    @pl.when(cid == 0)
    def _():
        pltpu.sync_copy(ldeg_v.at[pl.ds(sid * RPT, RPT)], rdeg_hbm.at[pl.ds(sid * RPT, RPT)])

    @pl.loop(0, NCHUNK)
    def _g(j2):
        for k in range(CHUNK // 16):
            idx = src_v[cid * NCHUNK + j2, pl.ds(k * 16, 16)]
            alpha_v[pl.ds(j2 * CHUNK + k * 16, 16)] = plsc.load_gather(ldeg_v, [idx])

    pltpu.sync_copy(alpha_v, alpha_hbm.at[pl.ds(wid * EPW, EPW)])


def _scatter(e4, m0, m1, zeros_nd, zeros_n):
    f = pl.kernel(
        _scatter_body,
        out_type=(
            jax.ShapeDtypeStruct((NP, D), jnp.float32),
            jax.ShapeDtypeStruct((E,), jnp.float32),
            jax.ShapeDtypeStruct((NP,), jnp.float32),
        ),
        mesh=_mesh(),
        scratch_types=[
            pltpu.VMEM((NCH2, CHUNK), jnp.int32),
            pltpu.VMEM((NCH2, CHUNK), jnp.int32),
            pltpu.VMEM((CHUNK, DH), jnp.float32),
            pltpu.VMEM((CHUNK, DH), jnp.float32),
            pltpu.VMEM((CHUNK,), jnp.float32),
            pltpu.VMEM((NP,), jnp.float32),
            pltpu.VMEM((EPW,), jnp.float32),
            pltpu.VMEM_SHARED((NP, DH), jnp.float32),
            pltpu.VMEM_SHARED((NP,), jnp.float32),
            pltpu.SemaphoreType.DMA,
            pltpu.SemaphoreType.DMA,
            pltpu.SemaphoreType.DMA,
            pltpu.SemaphoreType.DMA,
            pltpu.SemaphoreType.DMA,
        ],
        compiler_params=pltpu.CompilerParams(
            use_tc_tiling_on_sc=False, needs_layout_passes=False
        ),
    )
    return f(e4, m0, m1, zeros_nd, zeros_n)


def _ln_body(z_ref, agg_ref, rd_ref, g_ref, b_ref, out_ref):
    x = z_ref[...] + agg_ref[...] * rd_ref[...]
    mean = jnp.mean(x, axis=1, keepdims=True)
    xc = x - mean
    var = jnp.mean(xc * xc, axis=1, keepdims=True)
    out_ref[...] = xc * lax.rsqrt(var + 1e-5) * g_ref[...] + b_ref[...]


def _ln(z, agg_p, rdeg, gamma, beta):
    rd = rdeg.reshape(NP, 1)
    return pl.pallas_call(
        _ln_body,
        grid=(N // BLK,),
        in_specs=[
            pl.BlockSpec((BLK, D), lambda i: (i, 0)),
            pl.BlockSpec((BLK, D), lambda i: (i, 0)),
            pl.BlockSpec((BLK, 1), lambda i: (i, 0)),
            pl.BlockSpec((1, D), lambda i: (0, 0)),
            pl.BlockSpec((1, D), lambda i: (0, 0)),
        ],
        out_specs=pl.BlockSpec((BLK, D), lambda i: (i, 0)),
        out_shape=jax.ShapeDtypeStruct((N, D), jnp.float32),
    )(z, agg_p, rd, gamma, beta)


def kernel(z, edge_index, W1, b1, W2, b2, gamma, beta):
    e4 = edge_index.astype(jnp.int32).reshape(2, NS, NCH2, CHUNK)
    zeros_nd = jnp.zeros((NP, DH), jnp.float32)
    zeros_n = jnp.zeros((NP,), jnp.float32)

    m0, m1 = _mlp(z, W1, b1.reshape(1, D), W2, b2.reshape(1, D))
    agg_p, alpha, rdeg = _scatter(e4, m0, m1, zeros_nd, zeros_n)
    out = _ln(z, agg_p, rdeg, gamma.reshape(1, D), beta.reshape(1, D))
    return (out, alpha)
